# Initial kernel scaffold; baseline (speedup 1.0000x reference)
import jax
import jax.numpy as jnp
from jax.experimental import pallas as pl


def kernel(z, codebook):
    raise NotImplementedError("write your pallas kernel here")


# speedup vs baseline: 1.0938x; 1.0938x over previous
import functools

import jax
import jax.numpy as jnp
from jax import lax
from jax.experimental import pallas as pl
from jax.experimental.pallas import tpu as pltpu
from jax.experimental.pallas import tpu_sc as plsc

_B, _T, _D, _K = 32, 576, 64, 8192
_N = _B * _T
_TILE_T = 256
_TC_TILES = _N // _TILE_T
_SUB = 256
_NW = 32
_RPW = _N // _NW
_ICH = 96
_NCH = _RPW // _ICH


def _cbnorm_body(cb_ref, cbn_ref, cbnsq_ref):
    x = cb_ref[...]
    n = jnp.sqrt(jnp.sum(x * x, axis=-1, keepdims=True))
    cbn = x / jnp.clip(n, 1e-12)
    cbn_ref[...] = cbn
    cbnsq_ref[...] = jnp.sum(cbn * cbn, axis=-1, keepdims=True)


def _argmin_body(z_ref, cbn_ref, cbnsq_ref, idx_ref):
    zt = z_ref[...]
    nrm = jnp.sqrt(jnp.sum(zt * zt, axis=-1, keepdims=True))
    zn = zt / jnp.clip(nrm, 1e-12)
    znsq = jnp.sum(zn * zn, axis=-1, keepdims=True)
    znsq_row = jnp.transpose(znsq, (1, 0))
    bv = None
    bi = None
    for s in range(_K // _SUB):
        cbs = cbn_ref[pl.ds(s * _SUB, _SUB), :]
        nsq = cbnsq_ref[pl.ds(s * _SUB, _SUB), :]
        dot = lax.dot_general(cbs, zn, (((1,), (1,)), ((), ())))
        d = (znsq_row + nsq) - 2.0 * dot
        m = jnp.min(d, axis=0, keepdims=True)
        ids = lax.broadcasted_iota(jnp.int32, d.shape, 0) + (s * _SUB)
        ii = jnp.min(jnp.where(d <= m, ids, _K), axis=0, keepdims=True)
        if bv is None:
            bv, bi = m, ii
        else:
            take = m < bv
            bv = jnp.where(take, m, bv)
            bi = jnp.where(take, ii, bi)
    idx_ref[...] = bi.reshape(1, 1, _TILE_T)


def _sc_body(cbn_hbm, idx_hbm, z_hbm, out_hbm, lp_hbm,
             idx_v, rows_v, z_v, lp_v, sem):
    wid = lax.axis_index("s") * 2 + lax.axis_index("c")
    base = wid * _RPW
    pltpu.sync_copy(idx_hbm.at[wid], idx_v)
    copies = []
    for j in range(_NCH):
        copies.append(pltpu.async_copy(
            cbn_hbm.at[idx_v.at[j]],
            rows_v.at[pl.ds(j * _ICH, _ICH)], sem))
    pltpu.sync_copy(z_hbm.at[pl.ds(base, _RPW)], z_v)
    for c in copies:
        c.wait()

    def row(r, acc):
        for cc in range(_D // 16):
            sl = pl.ds(16 * cc, 16)
            q = rows_v[r, sl]
            zz = z_v[r, sl]
            t = q - zz
            rows_v[r, sl] = zz + t
            acc = acc + t * t
        return acc

    acc = lax.fori_loop(0, _RPW, row, jnp.zeros((16,), jnp.float32))
    lp_v[...] = acc
    pltpu.sync_copy(rows_v, out_hbm.at[pl.ds(base, _RPW)])
    pltpu.sync_copy(lp_v, lp_hbm.at[wid])


def _cbnorm_call(codebook):
    return pl.pallas_call(
        _cbnorm_body,
        out_shape=(jax.ShapeDtypeStruct((_K, _D), jnp.float32),
                   jax.ShapeDtypeStruct((_K, 1), jnp.float32)),
    )(codebook)


def _argmin_call(zf, cbn, cbnsq_row):
    return pl.pallas_call(
        _argmin_body,
        grid=(_TC_TILES,),
        in_specs=[
            pl.BlockSpec((_TILE_T, _D), lambda t: (t, 0)),
            pl.BlockSpec((_K, _D), lambda t: (0, 0)),
            pl.BlockSpec((_K, 1), lambda t: (0, 0)),
        ],
        out_specs=pl.BlockSpec((1, 1, _TILE_T), lambda t: (t, 0, 0)),
        out_shape=jax.ShapeDtypeStruct((_TC_TILES, 1, _TILE_T), jnp.int32),
    )(zf, cbn, cbnsq_row)


def _sc_call(cbn, idx3, zf):
    mesh = plsc.VectorSubcoreMesh(core_axis_name="c", subcore_axis_name="s")
    f = pl.kernel(
        _sc_body,
        out_type=(jax.ShapeDtypeStruct((_N, _D), jnp.float32),
                  jax.ShapeDtypeStruct((_NW, 16), jnp.float32)),
        mesh=mesh,
        scratch_types=[
            pltpu.VMEM((_NCH, _ICH), jnp.int32),
            pltpu.VMEM((_RPW, _D), jnp.float32),
            pltpu.VMEM((_RPW, _D), jnp.float32),
            pltpu.VMEM((16,), jnp.float32),
            pltpu.SemaphoreType.DMA,
        ],
        compiler_params=pltpu.CompilerParams(use_tc_tiling_on_sc=False),
    )
    return f(cbn, idx3, zf)


def kernel(z, codebook):
    zf = z.reshape(_N, _D)
    cbn, cbnsq = _cbnorm_call(codebook)
    idx2 = _argmin_call(zf, cbn, cbnsq)
    idx = idx2.reshape(_N)
    out, lp = _sc_call(cbn, idx.reshape(_NW, _NCH, _ICH), zf)
    m = jnp.sum(lp) / (_N * _D)
    loss = m + 0.33 * m
    return out.reshape(z.shape), loss, idx


# speedup vs baseline: 1.1978x; 1.0950x over previous
import functools

import jax
import jax.numpy as jnp
from jax import lax
from jax.experimental import pallas as pl
from jax.experimental.pallas import tpu as pltpu
from jax.experimental.pallas import tpu_sc as plsc

_B, _T, _D, _K = 32, 576, 64, 8192
_N = _B * _T
_TILE_T = 256
_TC_TILES = _N // _TILE_T
_SUB = 256
_NW = 32
_RPW = _N // _NW
_ICH = 96
_NCH = _RPW // _ICH


def _cbnorm_body(cb_ref, cbn_ref, cbnsq_ref):
    x = cb_ref[...]
    n = jnp.sqrt(jnp.sum(x * x, axis=-1, keepdims=True))
    cbn = x / jnp.clip(n, 1e-12)
    cbn_ref[...] = cbn
    cbnsq_ref[...] = jnp.sum(cbn * cbn, axis=-1, keepdims=True)


def _argmin_body(z_ref, cbn_ref, cbnsq_ref, idx_ref):
    zt = z_ref[...]
    nrm = jnp.sqrt(jnp.sum(zt * zt, axis=-1, keepdims=True))
    zn = zt / jnp.clip(nrm, 1e-12)
    znsq = jnp.sum(zn * zn, axis=-1, keepdims=True)
    znsq_row = jnp.transpose(znsq, (1, 0))
    bv = None
    bi = None
    ids0 = lax.broadcasted_iota(
        jnp.int32, (_SUB, _TILE_T), 0).astype(jnp.float32)
    for s in range(_K // _SUB):
        cbs = cbn_ref[pl.ds(s * _SUB, _SUB), :]
        nsq = cbnsq_ref[pl.ds(s * _SUB, _SUB), :]
        dot = lax.dot_general(cbs, zn, (((1,), (1,)), ((), ())))
        d = (znsq_row + nsq) - 2.0 * dot
        m = jnp.min(d, axis=0, keepdims=True)
        ids = ids0 + jnp.float32(s * _SUB)
        ii = jnp.min(jnp.where(d <= m, ids, jnp.float32(_K)),
                     axis=0, keepdims=True)
        if bv is None:
            bv, bi = m, ii
        else:
            take = m < bv
            bv = jnp.where(take, m, bv)
            bi = jnp.where(take, ii, bi)
    idx_ref[...] = bi.astype(jnp.int32).reshape(1, 1, _TILE_T)


def _sc_body(cbn_hbm, idx_hbm, z_hbm, out_hbm, lp_hbm,
             idx_v, rows_v, z_v, lp_v, sem):
    wid = lax.axis_index("s") * 2 + lax.axis_index("c")
    base = wid * _RPW
    pltpu.sync_copy(idx_hbm.at[wid], idx_v)
    copies = []
    for j in range(_NCH):
        copies.append(pltpu.async_copy(
            cbn_hbm.at[idx_v.at[j]],
            rows_v.at[pl.ds(j * _ICH, _ICH)], sem))
    pltpu.sync_copy(z_hbm.at[pl.ds(base, _RPW)], z_v)
    for c in copies:
        c.wait()

    def row(r, acc):
        for cc in range(_D // 16):
            sl = pl.ds(16 * cc, 16)
            q = rows_v[r, sl]
            zz = z_v[r, sl]
            t = q - zz
            rows_v[r, sl] = zz + t
            acc = acc + t * t
        return acc

    acc = lax.fori_loop(0, _RPW, row, jnp.zeros((16,), jnp.float32))
    lp_v[...] = acc
    pltpu.sync_copy(rows_v, out_hbm.at[pl.ds(base, _RPW)])
    pltpu.sync_copy(lp_v, lp_hbm.at[wid])


def _cbnorm_call(codebook):
    return pl.pallas_call(
        _cbnorm_body,
        out_shape=(jax.ShapeDtypeStruct((_K, _D), jnp.float32),
                   jax.ShapeDtypeStruct((_K, 1), jnp.float32)),
    )(codebook)


def _argmin_call(zf, cbn, cbnsq_row):
    return pl.pallas_call(
        _argmin_body,
        grid=(_TC_TILES,),
        in_specs=[
            pl.BlockSpec((_TILE_T, _D), lambda t: (t, 0)),
            pl.BlockSpec((_K, _D), lambda t: (0, 0)),
            pl.BlockSpec((_K, 1), lambda t: (0, 0)),
        ],
        out_specs=pl.BlockSpec((1, 1, _TILE_T), lambda t: (t, 0, 0)),
        out_shape=jax.ShapeDtypeStruct((_TC_TILES, 1, _TILE_T), jnp.int32),
    )(zf, cbn, cbnsq_row)


def _sc_call(cbn, idx3, zf):
    mesh = plsc.VectorSubcoreMesh(core_axis_name="c", subcore_axis_name="s")
    f = pl.kernel(
        _sc_body,
        out_type=(jax.ShapeDtypeStruct((_N, _D), jnp.float32),
                  jax.ShapeDtypeStruct((_NW, 16), jnp.float32)),
        mesh=mesh,
        scratch_types=[
            pltpu.VMEM((_NCH, _ICH), jnp.int32),
            pltpu.VMEM((_RPW, _D), jnp.float32),
            pltpu.VMEM((_RPW, _D), jnp.float32),
            pltpu.VMEM((16,), jnp.float32),
            pltpu.SemaphoreType.DMA,
        ],
        compiler_params=pltpu.CompilerParams(use_tc_tiling_on_sc=False),
    )
    return f(cbn, idx3, zf)


def kernel(z, codebook):
    zf = z.reshape(_N, _D)
    cbn, cbnsq = _cbnorm_call(codebook)
    idx2 = _argmin_call(zf, cbn, cbnsq)
    idx = idx2.reshape(_N)
    out, lp = _sc_call(cbn, idx.reshape(_NW, _NCH, _ICH), zf)
    m = jnp.sum(lp) / (_N * _D)
    loss = m + 0.33 * m
    return out.reshape(z.shape), loss, idx
